# Initial kernel scaffold; baseline (speedup 1.0000x reference)
import jax
import jax.numpy as jnp
from jax.experimental import pallas as pl


def kernel(x, edge_index, edge_attr, params):
    raise NotImplementedError("write your pallas kernel here")


# speedup vs baseline: 2.3868x; 2.3868x over previous
import functools

import jax
import jax.numpy as jnp
from jax import lax
from jax.experimental import pallas as pl
from jax.experimental.pallas import tpu as pltpu
from jax.experimental.pallas import tpu_sc as plsc

_N = 10000
_E = 160000
_ND = 256
_ED = 256

_BN = 1000
_BE = 1000

_EPS = 1e-5


def _ln(y):
    mu = jnp.mean(y, axis=-1, keepdims=True)
    var = jnp.mean((y - mu) ** 2, axis=-1, keepdims=True)
    return (y - mu) * lax.rsqrt(var + _EPS)


def _mlp_ln_body(x_ref, w1_ref, b1_ref, w2_ref, b2_ref, o_ref):
    h = jax.nn.silu(jnp.dot(x_ref[...], w1_ref[...]) + b1_ref[...])
    o_ref[...] = _ln(jnp.dot(h, w2_ref[...]) + b2_ref[...])


def _mlp_ln(xin, w1, b1, w2, b2, block):
    rows, d_in = xin.shape
    hdim = w1.shape[1]
    d_out = w2.shape[1]
    return pl.pallas_call(
        _mlp_ln_body,
        grid=(rows // block,),
        in_specs=[
            pl.BlockSpec((block, d_in), lambda i: (i, 0)),
            pl.BlockSpec((d_in, hdim), lambda i: (0, 0)),
            pl.BlockSpec((1, hdim), lambda i: (0, 0)),
            pl.BlockSpec((hdim, d_out), lambda i: (0, 0)),
            pl.BlockSpec((1, d_out), lambda i: (0, 0)),
        ],
        out_specs=pl.BlockSpec((block, d_out), lambda i: (i, 0)),
        out_shape=jax.ShapeDtypeStruct((rows, d_out), jnp.float32),
    )(xin, w1, b1.reshape(1, -1), w2, b2.reshape(1, -1))


def _edge_update_body(g_ref, he_ref, w1e_ref, b1_ref, w2_ref, b2_ref, o_ref):
    he = he_ref[...]
    h = jax.nn.silu(g_ref[...] + jnp.dot(he, w1e_ref[...]) + b1_ref[...])
    o_ref[...] = _ln(jnp.dot(h, w2_ref[...]) + b2_ref[...]) + he


def _edge_update(g, h_edge, w1e, b1, w2, b2):
    return pl.pallas_call(
        _edge_update_body,
        grid=(_E // _BE,),
        in_specs=[
            pl.BlockSpec((_BE, _ED), lambda i: (i, 0)),
            pl.BlockSpec((_BE, _ED), lambda i: (i, 0)),
            pl.BlockSpec((_ED, _ED), lambda i: (0, 0)),
            pl.BlockSpec((1, _ED), lambda i: (0, 0)),
            pl.BlockSpec((_ED, _ED), lambda i: (0, 0)),
            pl.BlockSpec((1, _ED), lambda i: (0, 0)),
        ],
        out_specs=pl.BlockSpec((_BE, _ED), lambda i: (i, 0)),
        out_shape=jax.ShapeDtypeStruct((_E, _ED), jnp.float32),
    )(g, h_edge, w1e, b1.reshape(1, -1), w2, b2.reshape(1, -1))


def _node_update_body(hn_ref, ag_ref, w1a_ref, w1b_ref, b1_ref, w2_ref,
                      b2_ref, o_ref):
    hn = hn_ref[...]
    h = jax.nn.silu(jnp.dot(hn, w1a_ref[...]) + jnp.dot(ag_ref[...], w1b_ref[...])
                    + b1_ref[...])
    o_ref[...] = _ln(jnp.dot(h, w2_ref[...]) + b2_ref[...]) + hn


def _node_update(h_node, agg, w1a, w1b, b1, w2, b2):
    return pl.pallas_call(
        _node_update_body,
        grid=(_N // _BN,),
        in_specs=[
            pl.BlockSpec((_BN, _ND), lambda i: (i, 0)),
            pl.BlockSpec((_BN, _ED), lambda i: (i, 0)),
            pl.BlockSpec((_ND, _ND), lambda i: (0, 0)),
            pl.BlockSpec((_ED, _ND), lambda i: (0, 0)),
            pl.BlockSpec((1, _ND), lambda i: (0, 0)),
            pl.BlockSpec((_ND, _ND), lambda i: (0, 0)),
            pl.BlockSpec((1, _ND), lambda i: (0, 0)),
        ],
        out_specs=pl.BlockSpec((_BN, _ND), lambda i: (i, 0)),
        out_shape=jax.ShapeDtypeStruct((_N, _ND), jnp.float32),
    )(h_node, agg, w1a, w1b, b1.reshape(1, -1), w2, b2.reshape(1, -1))


def _pq_body(hn_ref, w1s_ref, w1d_ref, p_ref, q_ref):
    hn = hn_ref[...]
    p_ref[...] = jnp.dot(hn, w1s_ref[...])
    q_ref[...] = jnp.dot(hn, w1d_ref[...])


def _pq(h_node, w1s, w1d):
    return pl.pallas_call(
        _pq_body,
        grid=(_N // _BN,),
        in_specs=[
            pl.BlockSpec((_BN, _ND), lambda i: (i, 0)),
            pl.BlockSpec((_ND, _ED), lambda i: (0, 0)),
            pl.BlockSpec((_ND, _ED), lambda i: (0, 0)),
        ],
        out_specs=[
            pl.BlockSpec((_BN, _ED), lambda i: (i, 0)),
            pl.BlockSpec((_BN, _ED), lambda i: (i, 0)),
        ],
        out_shape=[
            jax.ShapeDtypeStruct((_N, _ED), jnp.float32),
            jax.ShapeDtypeStruct((_N, _ED), jnp.float32),
        ],
    )(h_node, w1s, w1d)


def _decoder_body(hn_ref, w1_ref, b1_ref, w2_ref, b2_ref, o_ref):
    h = jax.nn.silu(jnp.dot(hn_ref[...], w1_ref[...]) + b1_ref[...])
    o_ref[...] = jnp.dot(h, w2_ref[...]) + b2_ref[...]


def _decoder(h_node, w1, b1, w2p, b2p):
    d_out = w2p.shape[1]
    return pl.pallas_call(
        _decoder_body,
        grid=(_N // _BN,),
        in_specs=[
            pl.BlockSpec((_BN, _ND), lambda i: (i, 0)),
            pl.BlockSpec((_ND, _ND), lambda i: (0, 0)),
            pl.BlockSpec((1, _ND), lambda i: (0, 0)),
            pl.BlockSpec((_ND, d_out), lambda i: (0, 0)),
            pl.BlockSpec((1, d_out), lambda i: (0, 0)),
        ],
        out_specs=pl.BlockSpec((_BN, d_out), lambda i: (i, 0)),
        out_shape=jax.ShapeDtypeStruct((_N, d_out), jnp.float32),
    )(h_node, w1, b1.reshape(1, -1), w2p, b2p.reshape(1, -1))


_INFO = plsc.get_sparse_core_info()
_NC = _INFO.num_cores
_NS = _INFO.num_subcores
_LN = _INFO.num_lanes
_NW = _NC * _NS

_GC = 128
_NCHUNK = _E // _GC
_GPW = -(-_NCHUNK // _NW)
_CPT = -(-_NCHUNK // _NS)
_HC = _ED // _NC
_CO = 200
_NROWCH = _N // _CO
_RPT = -(-_NROWCH // _NS)


def _sc_gather_sum(p, q, src, dst):
    mesh = plsc.VectorSubcoreMesh(core_axis_name="c", subcore_axis_name="s")

    @functools.partial(
        pl.kernel,
        mesh=mesh,
        out_type=jax.ShapeDtypeStruct((_E, _ED), jnp.float32),
        scratch_types=[
            pltpu.VMEM((_GC,), jnp.int32),
            pltpu.VMEM((_GC,), jnp.int32),
            pltpu.VMEM((_GC, _ED), jnp.float32),
            pltpu.VMEM((_GC, _ED), jnp.float32),
            pltpu.SemaphoreType.DMA,
            pltpu.SemaphoreType.DMA,
        ],
    )
    def k(p_hbm, q_hbm, src_hbm, dst_hbm, out_hbm, sidx, didx, bufa, bufb,
          sem1, sem2):
        wid = lax.axis_index("s") * _NC + lax.axis_index("c")

        def step(g, carry):
            chunk = g * _NW + wid

            @pl.when(chunk < _NCHUNK)
            def _():
                e0 = chunk * _GC
                pltpu.sync_copy(src_hbm.at[pl.ds(e0, _GC)], sidx)
                pltpu.sync_copy(dst_hbm.at[pl.ds(e0, _GC)], didx)
                cpa = pltpu.async_copy(p_hbm.at[sidx], bufa, sem1)
                cpb = pltpu.async_copy(q_hbm.at[didx], bufb, sem2)
                cpa.wait()
                cpb.wait()

                def add_row(r, c2):
                    for u in range(_ED // _LN):
                        sl = pl.ds(u * _LN, _LN)
                        bufa[r, sl] = bufa[r, sl] + bufb[r, sl]
                    return c2

                lax.fori_loop(0, _GC, add_row, 0)
                pltpu.sync_copy(bufa, out_hbm.at[pl.ds(e0, _GC)])

            return carry

        lax.fori_loop(0, _GPW, step, 0)

    return k(p, q, src, dst)


def _sc_segment_sum(he, dst):
    mesh = plsc.VectorSubcoreMesh(core_axis_name="c", subcore_axis_name="s")

    @functools.partial(
        pl.kernel,
        mesh=mesh,
        out_type=jax.ShapeDtypeStruct((_NC, _N, _HC), jnp.float32),
        scratch_types=[
            pltpu.VMEM((_GC,), jnp.int32),
            pltpu.VMEM((_GC, _HC), jnp.float32),
            pltpu.VMEM((_CO, _HC), jnp.float32),
            pltpu.VMEM_SHARED((_N, _HC), jnp.float32),
        ],
    )
    def k(he_hbm, dst_hbm, out_hbm, idx, data, cobuf, table):
        c = lax.axis_index("c")
        s = lax.axis_index("s")

        def zrow(r, carry):
            for u in range(_HC // _LN):
                cobuf[r, pl.ds(u * _LN, _LN)] = jnp.zeros((_LN,), jnp.float32)
            return carry

        lax.fori_loop(0, _CO, zrow, 0)

        def zchunk(j, carry):
            rc = j * _NS + s

            @pl.when(rc < _NROWCH)
            def _():
                pltpu.sync_copy(cobuf, table.at[pl.ds(rc * _CO, _CO)])

            return carry

        lax.fori_loop(0, _RPT, zchunk, 0)
        plsc.subcore_barrier()

        col0 = c * _HC

        def step(g, carry):
            chunk = g * _NS + s

            @pl.when(chunk < _NCHUNK)
            def _():
                e0 = chunk * _GC
                pltpu.sync_copy(dst_hbm.at[pl.ds(e0, _GC)], idx)
                pltpu.sync_copy(he_hbm.at[pl.ds(e0, _GC), pl.ds(col0, _HC)],
                                data)
                pltpu.sync_copy(data, table.at[idx], add=True)

            return carry

        lax.fori_loop(0, _CPT, step, 0)
        plsc.subcore_barrier()

        def cochunk(j, carry):
            rc = j * _NS + s

            @pl.when(rc < _NROWCH)
            def _():
                pltpu.sync_copy(table.at[pl.ds(rc * _CO, _CO)], cobuf)
                pltpu.sync_copy(cobuf, out_hbm.at[c, pl.ds(rc * _CO, _CO)])

            return carry

        lax.fori_loop(0, _RPT, cochunk, 0)

    halves = k(he, dst)
    return jnp.concatenate([halves[0], halves[1]], axis=1)


def kernel(x, edge_index, edge_attr, params):
    src = edge_index[0]
    dst = edge_index[1]

    en = params["enc_n"]
    ee = params["enc_e"]
    h_node = _mlp_ln(x, en[0], en[1], en[2], en[3], _BN)
    h_edge = _mlp_ln(edge_attr, ee[0], ee[1], ee[2], ee[3], _BE)

    for cp in params["convs"]:
        w1, b1, w2, b2 = cp["edge"]
        w1s = w1[:_ND]
        w1d = w1[_ND:2 * _ND]
        w1e = w1[2 * _ND:]
        p, q = _pq(h_node, w1s, w1d)
        g = _sc_gather_sum(p, q, src, dst)
        h_edge = _edge_update(g, h_edge, w1e, b1, w2, b2)
        agg = _sc_segment_sum(h_edge, dst)
        nw1, nb1, nw2, nb2 = cp["node"]
        h_node = _node_update(h_node, agg, nw1[:_ND], nw1[_ND:], nb1, nw2, nb2)

    ow1, ob1, ow2, ob2 = params["out"]
    d_out = ow2.shape[1]
    w2p = jnp.pad(ow2, ((0, 0), (0, 128 - d_out)))
    b2p = jnp.pad(ob2, (0, 128 - d_out))
    out = _decoder(h_node, ow1, ob1, w2p, b2p)
    return out[:, :d_out]


# speedup vs baseline: 4.6245x; 1.9376x over previous
import functools

import jax
import jax.numpy as jnp
from jax import lax
from jax.experimental import pallas as pl
from jax.experimental.pallas import tpu as pltpu
from jax.experimental.pallas import tpu_sc as plsc

_N = 10000
_E = 160000
_EH = _E // 2
_ND = 256
_ED = 256

_BN = 2000
_BE = 2000

_EPS = 1e-5


def _ln(y):
    mu = jnp.mean(y, axis=-1, keepdims=True)
    var = jnp.mean((y - mu) ** 2, axis=-1, keepdims=True)
    return (y - mu) * lax.rsqrt(var + _EPS)


def _bdot(a, b):
    return jnp.dot(a, b, preferred_element_type=jnp.float32)


def _pack_bf16(y):
    d2 = y.shape[1] // 2
    u = lax.bitcast_convert_type(y, jnp.uint32)
    rnd = (u + jnp.uint32(0x7FFF) + ((u >> 16) & jnp.uint32(1))) >> 16
    w = rnd[:, :d2] | (rnd[:, d2:] << 16)
    return lax.bitcast_convert_type(w, jnp.int32)


def _unpack_bf16(w):
    u = lax.bitcast_convert_type(w, jnp.uint32)
    ylo = lax.bitcast_convert_type(u << 16, jnp.float32)
    yhi = lax.bitcast_convert_type(u & jnp.uint32(0xFFFF0000), jnp.float32)
    return jnp.concatenate([ylo, yhi], axis=1)


def _mlp_ln_body(x_ref, w1_ref, b1_ref, w2_ref, b2_ref, o_ref):
    h = jax.nn.silu(_bdot(x_ref[...], w1_ref[...]) + b1_ref[...])
    o_ref[...] = _ln(_bdot(h, w2_ref[...]) + b2_ref[...])


def _mlp_ln(xin, w1, b1, w2, b2, block, row_off=0, rows_out=None):
    rows, d_in = xin.shape
    if rows_out is None:
        rows_out = rows
    hdim = w1.shape[1]
    d_out = w2.shape[1]
    off = row_off // block
    return pl.pallas_call(
        _mlp_ln_body,
        grid=(rows_out // block,),
        in_specs=[
            pl.BlockSpec((block, d_in), lambda i: (i + off, 0)),
            pl.BlockSpec((d_in, hdim), lambda i: (0, 0)),
            pl.BlockSpec((1, hdim), lambda i: (0, 0)),
            pl.BlockSpec((hdim, d_out), lambda i: (0, 0)),
            pl.BlockSpec((1, d_out), lambda i: (0, 0)),
        ],
        out_specs=pl.BlockSpec((block, d_out), lambda i: (i, 0)),
        out_shape=jax.ShapeDtypeStruct((rows_out, d_out), jnp.float32),
    )(xin, w1, b1.reshape(1, -1), w2, b2.reshape(1, -1))


def _edge_update_body(g_ref, he_ref, w1e_ref, b1_ref, w2_ref, b2_ref,
                      o_ref):
    he = he_ref[...]
    g = _unpack_bf16(g_ref[...])
    h = jax.nn.silu(g + _bdot(he, w1e_ref[...]) + b1_ref[...])
    o_ref[...] = _ln(_bdot(h, w2_ref[...]) + b2_ref[...]) + he


def _edge_update(g, h_edge, w1e, b1, w2, b2):
    rows = g.shape[0]
    return pl.pallas_call(
        _edge_update_body,
        grid=(rows // _BE,),
        in_specs=[
            pl.BlockSpec((_BE, _ED // 2), lambda i: (i, 0)),
            pl.BlockSpec((_BE, _ED), lambda i: (i, 0)),
            pl.BlockSpec((_ED, _ED), lambda i: (0, 0)),
            pl.BlockSpec((1, _ED), lambda i: (0, 0)),
            pl.BlockSpec((_ED, _ED), lambda i: (0, 0)),
            pl.BlockSpec((1, _ED), lambda i: (0, 0)),
        ],
        out_specs=pl.BlockSpec((_BE, _ED), lambda i: (i, 0)),
        out_shape=jax.ShapeDtypeStruct((rows, _ED), jnp.float32),
    )(g, h_edge, w1e, b1.reshape(1, -1), w2, b2.reshape(1, -1))


def _node_update_body(hn_ref, a1_ref, a2_ref, b1_ref_, b2_ref_, w1a_ref,
                      w1b1_ref, w1b2_ref, b1_ref, w2_ref, b2_ref, o_ref):
    hn = hn_ref[...]
    ag1 = a1_ref[...] + b1_ref_[...]
    ag2 = a2_ref[...] + b2_ref_[...]
    h = jax.nn.silu(_bdot(hn, w1a_ref[...]) + _bdot(ag1, w1b1_ref[...])
                    + _bdot(ag2, w1b2_ref[...]) + b1_ref[...])
    o_ref[...] = _ln(_bdot(h, w2_ref[...]) + b2_ref[...]) + hn


def _node_update_pq_body(hn_ref, a1_ref, a2_ref, b1_ref_, b2_ref_, w1a_ref,
                         w1b1_ref, w1b2_ref, b1_ref, w2_ref, b2_ref,
                         w1s_ref, w1d_ref, o_ref, p_ref, q_ref):
    hn = hn_ref[...]
    ag1 = a1_ref[...] + b1_ref_[...]
    ag2 = a2_ref[...] + b2_ref_[...]
    h = jax.nn.silu(_bdot(hn, w1a_ref[...]) + _bdot(ag1, w1b1_ref[...])
                    + _bdot(ag2, w1b2_ref[...]) + b1_ref[...])
    hn2 = _ln(_bdot(h, w2_ref[...]) + b2_ref[...]) + hn
    o_ref[...] = hn2
    p_ref[...] = _pack_bf16(_bdot(hn2, w1s_ref[...]))
    q_ref[...] = _pack_bf16(_bdot(hn2, w1d_ref[...]))


def _node_update(h_node, aggs, w1a, w1b1, w1b2, b1, w2, b2,
                 w1s=None, w1d=None):
    base_specs = [
        pl.BlockSpec((_BN, _ND), lambda i: (i, 0)),
        pl.BlockSpec((_BN, _HC), lambda i: (i, 0)),
        pl.BlockSpec((_BN, _HC), lambda i: (i, 0)),
        pl.BlockSpec((_BN, _HC), lambda i: (i, 0)),
        pl.BlockSpec((_BN, _HC), lambda i: (i, 0)),
        pl.BlockSpec((_ND, _ND), lambda i: (0, 0)),
        pl.BlockSpec((_HC, _ND), lambda i: (0, 0)),
        pl.BlockSpec((_HC, _ND), lambda i: (0, 0)),
        pl.BlockSpec((1, _ND), lambda i: (0, 0)),
        pl.BlockSpec((_ND, _ND), lambda i: (0, 0)),
        pl.BlockSpec((1, _ND), lambda i: (0, 0)),
    ]
    args = [h_node] + list(aggs) + [w1a, w1b1, w1b2, b1.reshape(1, -1), w2,
                                    b2.reshape(1, -1)]
    if w1s is None:
        return pl.pallas_call(
            _node_update_body,
            grid=(_N // _BN,),
            in_specs=base_specs,
            out_specs=pl.BlockSpec((_BN, _ND), lambda i: (i, 0)),
            out_shape=jax.ShapeDtypeStruct((_N, _ND), jnp.float32),
        )(*args)
    return pl.pallas_call(
        _node_update_pq_body,
        grid=(_N // _BN,),
        in_specs=base_specs + [
            pl.BlockSpec((_ND, _ED), lambda i: (0, 0)),
            pl.BlockSpec((_ND, _ED), lambda i: (0, 0)),
        ],
        out_specs=[
            pl.BlockSpec((_BN, _ND), lambda i: (i, 0)),
            pl.BlockSpec((_BN, _ED // 2), lambda i: (i, 0)),
            pl.BlockSpec((_BN, _ED // 2), lambda i: (i, 0)),
        ],
        out_shape=[
            jax.ShapeDtypeStruct((_N, _ND), jnp.float32),
            jax.ShapeDtypeStruct((_N, _ED // 2), jnp.int32),
            jax.ShapeDtypeStruct((_N, _ED // 2), jnp.int32),
        ],
    )(*(args + [w1s, w1d]))


def _pq_body(hn_ref, w1s_ref, w1d_ref, p_ref, q_ref):
    hn = hn_ref[...]
    p_ref[...] = _pack_bf16(_bdot(hn, w1s_ref[...]))
    q_ref[...] = _pack_bf16(_bdot(hn, w1d_ref[...]))


def _pq(h_node, w1s, w1d):
    return pl.pallas_call(
        _pq_body,
        grid=(_N // _BN,),
        in_specs=[
            pl.BlockSpec((_BN, _ND), lambda i: (i, 0)),
            pl.BlockSpec((_ND, _ED), lambda i: (0, 0)),
            pl.BlockSpec((_ND, _ED), lambda i: (0, 0)),
        ],
        out_specs=[
            pl.BlockSpec((_BN, _ED // 2), lambda i: (i, 0)),
            pl.BlockSpec((_BN, _ED // 2), lambda i: (i, 0)),
        ],
        out_shape=[
            jax.ShapeDtypeStruct((_N, _ED // 2), jnp.int32),
            jax.ShapeDtypeStruct((_N, _ED // 2), jnp.int32),
        ],
    )(h_node, w1s, w1d)


def _decoder_body(hn_ref, w1_ref, b1_ref, w2_ref, b2_ref, o_ref):
    h = jax.nn.silu(_bdot(hn_ref[...], w1_ref[...]) + b1_ref[...])
    o_ref[...] = _bdot(h, w2_ref[...]) + b2_ref[...]


def _decoder(h_node, w1, b1, w2p, b2p):
    d_out = w2p.shape[1]
    return pl.pallas_call(
        _decoder_body,
        grid=(_N // _BN,),
        in_specs=[
            pl.BlockSpec((_BN, _ND), lambda i: (i, 0)),
            pl.BlockSpec((_ND, _ND), lambda i: (0, 0)),
            pl.BlockSpec((1, _ND), lambda i: (0, 0)),
            pl.BlockSpec((_ND, d_out), lambda i: (0, 0)),
            pl.BlockSpec((1, d_out), lambda i: (0, 0)),
        ],
        out_specs=pl.BlockSpec((_BN, d_out), lambda i: (i, 0)),
        out_shape=jax.ShapeDtypeStruct((_N, d_out), jnp.float32),
    )(h_node, w1, b1.reshape(1, -1), w2p, b2p.reshape(1, -1))


_INFO = plsc.get_sparse_core_info()
_NC = _INFO.num_cores
_NS = _INFO.num_subcores
_LN = _INFO.num_lanes
_NW = _NC * _NS

_GC = 128
_HC = _ED // _NC
_CO = 80
_NROWCH = _N // _CO
_RPT = -(-_NROWCH // _NS)
_HW = _ED // 2


def _sc_gather_sum(p, q, src, dst, e_start, e_count):
    mesh = plsc.VectorSubcoreMesh(core_axis_name="c", subcore_axis_name="s")
    nchunk = e_count // _GC
    gpw = -(-nchunk // _NW)
    mask_c = jnp.int32(-65536)
    half_c = jnp.int32(32768)

    @functools.partial(
        pl.kernel,
        mesh=mesh,
        out_type=jax.ShapeDtypeStruct((e_count, _HW), jnp.int32),
        scratch_types=[
            pltpu.VMEM((2, _GC), jnp.int32),
            pltpu.VMEM((2, _GC), jnp.int32),
            pltpu.VMEM((2, _GC, _HW), jnp.int32),
            pltpu.VMEM((2, _GC, _HW), jnp.int32),
            pltpu.VMEM((2, _GC, _HW), jnp.int32),
            pltpu.SemaphoreType.DMA,
            pltpu.SemaphoreType.DMA,
            pltpu.SemaphoreType.DMA,
            pltpu.SemaphoreType.DMA,
            pltpu.SemaphoreType.DMA,
            pltpu.SemaphoreType.DMA,
        ],
    )
    def k(p_hbm, q_hbm, src_hbm, dst_hbm, out_hbm, sidx, didx, bufa, bufb,
          obuf, sa0, sa1, sb0, sb1, so0, so1):
        wid = lax.axis_index("s") * _NC + lax.axis_index("c")
        sas = (sa0, sa1)
        sbs = (sb0, sb1)
        sos = (so0, so1)

        def issue(g, slot):
            chunk = g * _NW + wid

            @pl.when(chunk < nchunk)
            def _():
                eg = e_start + chunk * _GC
                pltpu.sync_copy(src_hbm.at[pl.ds(eg, _GC)], sidx.at[slot])
                pltpu.sync_copy(dst_hbm.at[pl.ds(eg, _GC)], didx.at[slot])
                pltpu.async_copy(p_hbm.at[sidx.at[slot]], bufa.at[slot],
                                 sas[slot])
                pltpu.async_copy(q_hbm.at[didx.at[slot]], bufb.at[slot],
                                 sbs[slot])

        def process(g, slot):
            chunk = g * _NW + wid

            @pl.when(chunk < nchunk)
            def _():
                e0 = chunk * _GC
                pltpu.make_async_copy(p_hbm.at[sidx.at[slot]],
                                      bufa.at[slot], sas[slot]).wait()
                pltpu.make_async_copy(q_hbm.at[didx.at[slot]],
                                      bufb.at[slot], sbs[slot]).wait()

                def add_row(r, c2):
                    for u in range(_HW // _LN):
                        sl = pl.ds(u * _LN, _LN)
                        a = bufa[slot, r, sl]
                        b = bufb[slot, r, sl]
                        lo = (lax.bitcast_convert_type(
                                  lax.shift_left(a, 16), jnp.float32)
                              + lax.bitcast_convert_type(
                                  lax.shift_left(b, 16), jnp.float32))
                        hi = (lax.bitcast_convert_type(a & mask_c,
                                                       jnp.float32)
                              + lax.bitcast_convert_type(b & mask_c,
                                                         jnp.float32))
                        ulo = lax.bitcast_convert_type(lo, jnp.int32) + half_c
                        uhi = lax.bitcast_convert_type(hi, jnp.int32) + half_c
                        obuf[slot, r, sl] = (
                            lax.shift_right_logical(ulo, 16) | (uhi & mask_c))
                    return c2

                lax.fori_loop(0, _GC, add_row, 0)
                pltpu.async_copy(obuf.at[slot], out_hbm.at[pl.ds(e0, _GC)],
                                 sos[slot])

        def drainw(g, slot):
            chunk = g * _NW + wid

            @pl.when((g >= 0) & (chunk < nchunk))
            def _():
                e0 = chunk * _GC
                pltpu.make_async_copy(obuf.at[slot],
                                      out_hbm.at[pl.ds(e0, _GC)],
                                      sos[slot]).wait()

        issue(0, 0)
        issue(1, 1)

        def step(j, carry):
            g0 = j * 2
            drainw(g0 - 2, 0)
            process(g0, 0)
            issue(g0 + 2, 0)
            drainw(g0 - 1, 1)
            process(g0 + 1, 1)
            issue(g0 + 3, 1)
            return carry

        nsteps = -(-gpw // 2)
        lax.fori_loop(0, nsteps, step, 0)
        drainw(nsteps * 2 - 2, 0)
        drainw(nsteps * 2 - 1, 1)

    return k(p, q, src, dst)


def _sc_segment_sum(he, dst, e_start):
    mesh = plsc.VectorSubcoreMesh(core_axis_name="c", subcore_axis_name="s")
    e_count = he.shape[0]
    nchunk = e_count // _GC
    cpt = -(-nchunk // _NS)

    @functools.partial(
        pl.kernel,
        mesh=mesh,
        out_type=[jax.ShapeDtypeStruct((_N, _HC), jnp.float32),
                  jax.ShapeDtypeStruct((_N, _HC), jnp.float32)],
        scratch_types=[
            pltpu.VMEM((2, _GC), jnp.int32),
            pltpu.VMEM((2, _GC, _HC), jnp.float32),
            pltpu.VMEM((_CO, _HC), jnp.float32),
            pltpu.VMEM_SHARED((_N, _HC), jnp.float32),
            pltpu.SemaphoreType.DMA,
            pltpu.SemaphoreType.DMA,
            pltpu.SemaphoreType.DMA,
            pltpu.SemaphoreType.DMA,
            pltpu.SemaphoreType.DMA,
            pltpu.SemaphoreType.DMA,
        ],
    )
    def k(he_hbm, dst_hbm, o1_hbm, o2_hbm, idx, data, cobuf, table,
          li0, li1, ld0, ld1, ss0, ss1):
        c = lax.axis_index("c")
        s = lax.axis_index("s")
        lis = (li0, li1)
        lds = (ld0, ld1)
        sss = (ss0, ss1)
        col0 = c * _HC

        def load(g, slot):
            chunk = g * _NS + s

            @pl.when(chunk < nchunk)
            def _():
                e0 = chunk * _GC
                pltpu.async_copy(dst_hbm.at[pl.ds(e_start + e0, _GC)],
                                 idx.at[slot], lis[slot])
                pltpu.async_copy(he_hbm.at[pl.ds(e0, _GC), pl.ds(col0, _HC)],
                                 data.at[slot], lds[slot])

        def scat(g, slot):
            chunk = g * _NS + s

            @pl.when(chunk < nchunk)
            def _():
                e0 = chunk * _GC
                pltpu.make_async_copy(dst_hbm.at[pl.ds(e_start + e0, _GC)],
                                      idx.at[slot], lis[slot]).wait()
                pltpu.make_async_copy(
                    he_hbm.at[pl.ds(e0, _GC), pl.ds(col0, _HC)],
                    data.at[slot], lds[slot]).wait()
                pltpu.async_copy(data.at[slot], table.at[idx.at[slot]],
                                 sss[slot], add=True)

        def drains(g, slot):
            chunk = g * _NS + s

            @pl.when(chunk < nchunk)
            def _():
                pltpu.make_async_copy(data.at[slot],
                                      table.at[idx.at[slot]],
                                      sss[slot]).wait()

        load(0, 0)
        load(1, 1)

        def zrow(r, carry):
            for u in range(_HC // _LN):
                cobuf[r, pl.ds(u * _LN, _LN)] = jnp.zeros((_LN,), jnp.float32)
            return carry

        lax.fori_loop(0, _CO, zrow, 0)

        def zchunk(j, carry):
            rc = j * _NS + s

            @pl.when(rc < _NROWCH)
            def _():
                pltpu.sync_copy(cobuf, table.at[pl.ds(rc * _CO, _CO)])

            return carry

        lax.fori_loop(0, _RPT, zchunk, 0)
        plsc.subcore_barrier()

        def step(j, carry):
            g0 = j * 2
            scat(g0, 0)
            drains(g0, 0)
            load(g0 + 2, 0)
            scat(g0 + 1, 1)
            drains(g0 + 1, 1)
            load(g0 + 3, 1)
            return carry

        lax.fori_loop(0, -(-cpt // 2), step, 0)
        plsc.subcore_barrier()

        def cochunk(j, carry):
            rc = j * _NS + s

            @pl.when(rc < _NROWCH)
            def _():
                pltpu.sync_copy(table.at[pl.ds(rc * _CO, _CO)], cobuf)

                @pl.when(c == 0)
                def _():
                    pltpu.sync_copy(cobuf, o1_hbm.at[pl.ds(rc * _CO, _CO)])

                @pl.when(c == 1)
                def _():
                    pltpu.sync_copy(cobuf, o2_hbm.at[pl.ds(rc * _CO, _CO)])

            return carry

        lax.fori_loop(0, _RPT, cochunk, 0)

    return k(he, dst)


def kernel(x, edge_index, edge_attr, params):
    src = edge_index[0]
    dst = edge_index[1]

    en = params["enc_n"]
    ee = params["enc_e"]
    h_node = _mlp_ln(x, en[0], en[1], en[2], en[3], _BN)
    he_a = _mlp_ln(edge_attr, ee[0], ee[1], ee[2], ee[3], _BE,
                   row_off=0, rows_out=_EH)
    he_b = _mlp_ln(edge_attr, ee[0], ee[1], ee[2], ee[3], _BE,
                   row_off=_EH, rows_out=_EH)

    convs = params["convs"]
    ew = [cp["edge"] for cp in convs]
    p32, q32 = _pq(h_node, ew[0][0][:_ND], ew[0][0][_ND:2 * _ND])
    for i, cp in enumerate(convs):
        w1, b1, w2, b2 = ew[i]
        w1e = w1[2 * _ND:]
        ga = _sc_gather_sum(p32, q32, src, dst, 0, _EH)
        gb = _sc_gather_sum(p32, q32, src, dst, _EH, _EH)
        he_a = _edge_update(ga, he_a, w1e, b1, w2, b2)
        agg_a = _sc_segment_sum(he_a, dst, 0)
        he_b = _edge_update(gb, he_b, w1e, b1, w2, b2)
        agg_b = _sc_segment_sum(he_b, dst, _EH)
        nw1, nb1, nw2, nb2 = cp["node"]
        aggs = (agg_a[0], agg_a[1], agg_b[0], agg_b[1])
        if i + 1 < len(convs):
            nxt = ew[i + 1][0]
            h_node, p32, q32 = _node_update(
                h_node, aggs, nw1[:_ND], nw1[_ND:_ND + _HC],
                nw1[_ND + _HC:], nb1, nw2, nb2,
                w1s=nxt[:_ND], w1d=nxt[_ND:2 * _ND])
        else:
            h_node = _node_update(h_node, aggs, nw1[:_ND],
                                  nw1[_ND:_ND + _HC], nw1[_ND + _HC:],
                                  nb1, nw2, nb2)

    ow1, ob1, ow2, ob2 = params["out"]
    d_out = ow2.shape[1]
    w2p = jnp.pad(ow2, ((0, 0), (0, 128 - d_out)))
    b2p = jnp.pad(ob2, (0, 128 - d_out))
    out = _decoder(h_node, ow1, ob1, w2p, b2p)
    return out[:, :d_out]


# speedup vs baseline: 4.8112x; 1.0404x over previous
import functools

import jax
import jax.numpy as jnp
from jax import lax
from jax.experimental import pallas as pl
from jax.experimental.pallas import tpu as pltpu
from jax.experimental.pallas import tpu_sc as plsc

_N = 10000
_E = 160000
_EH = _E // 2
_ND = 256
_ED = 256

_BN = 2000
_BE = 4000

_EPS = 1e-5


def _ln(y):
    mu = jnp.mean(y, axis=-1, keepdims=True)
    var = jnp.mean((y - mu) ** 2, axis=-1, keepdims=True)
    return (y - mu) * lax.rsqrt(var + _EPS)


def _bdot(a, b):
    return jnp.dot(a, b, preferred_element_type=jnp.float32)


def _pack_bf16(y):
    d2 = y.shape[1] // 2
    u = lax.bitcast_convert_type(y, jnp.uint32)
    rnd = (u + jnp.uint32(0x7FFF) + ((u >> 16) & jnp.uint32(1))) >> 16
    w = rnd[:, :d2] | (rnd[:, d2:] << 16)
    return lax.bitcast_convert_type(w, jnp.int32)


def _unpack_bf16(w):
    u = lax.bitcast_convert_type(w, jnp.uint32)
    ylo = lax.bitcast_convert_type(u << 16, jnp.float32)
    yhi = lax.bitcast_convert_type(u & jnp.uint32(0xFFFF0000), jnp.float32)
    return jnp.concatenate([ylo, yhi], axis=1)


def _mlp_ln_body(x_ref, w1_ref, b1_ref, w2_ref, b2_ref, o_ref):
    h = jax.nn.silu(_bdot(x_ref[...], w1_ref[...]) + b1_ref[...])
    o_ref[...] = _ln(_bdot(h, w2_ref[...]) + b2_ref[...])


def _mlp_ln(xin, w1, b1, w2, b2, block, row_off=0, rows_out=None):
    rows, d_in = xin.shape
    if rows_out is None:
        rows_out = rows
    hdim = w1.shape[1]
    d_out = w2.shape[1]
    off = row_off // block
    return pl.pallas_call(
        _mlp_ln_body,
        grid=(rows_out // block,),
        in_specs=[
            pl.BlockSpec((block, d_in), lambda i: (i + off, 0)),
            pl.BlockSpec((d_in, hdim), lambda i: (0, 0)),
            pl.BlockSpec((1, hdim), lambda i: (0, 0)),
            pl.BlockSpec((hdim, d_out), lambda i: (0, 0)),
            pl.BlockSpec((1, d_out), lambda i: (0, 0)),
        ],
        out_specs=pl.BlockSpec((block, d_out), lambda i: (i, 0)),
        out_shape=jax.ShapeDtypeStruct((rows_out, d_out), jnp.float32),
    )(xin, w1, b1.reshape(1, -1), w2, b2.reshape(1, -1))


def _edge_update_body(g_ref, he_ref, w1e_ref, b1_ref, w2_ref, b2_ref,
                      o_ref):
    he = he_ref[...]
    g = _unpack_bf16(g_ref[...])
    h = jax.nn.silu(g + _bdot(he, w1e_ref[...]) + b1_ref[...])
    o_ref[...] = _ln(_bdot(h, w2_ref[...]) + b2_ref[...]) + he


def _edge_update(g, h_edge, w1e, b1, w2, b2):
    rows = g.shape[0]
    return pl.pallas_call(
        _edge_update_body,
        grid=(rows // _BE,),
        in_specs=[
            pl.BlockSpec((_BE, _ED // 2), lambda i: (i, 0)),
            pl.BlockSpec((_BE, _ED), lambda i: (i, 0)),
            pl.BlockSpec((_ED, _ED), lambda i: (0, 0)),
            pl.BlockSpec((1, _ED), lambda i: (0, 0)),
            pl.BlockSpec((_ED, _ED), lambda i: (0, 0)),
            pl.BlockSpec((1, _ED), lambda i: (0, 0)),
        ],
        out_specs=pl.BlockSpec((_BE, _ED), lambda i: (i, 0)),
        out_shape=jax.ShapeDtypeStruct((rows, _ED), jnp.float32),
    )(g, h_edge, w1e, b1.reshape(1, -1), w2, b2.reshape(1, -1))


def _node_update_body(hn_ref, a1_ref, a2_ref, b1_ref_, b2_ref_, w1a_ref,
                      w1b1_ref, w1b2_ref, b1_ref, w2_ref, b2_ref, o_ref):
    hn = hn_ref[...]
    ag1 = a1_ref[...] + b1_ref_[...]
    ag2 = a2_ref[...] + b2_ref_[...]
    h = jax.nn.silu(_bdot(hn, w1a_ref[...]) + _bdot(ag1, w1b1_ref[...])
                    + _bdot(ag2, w1b2_ref[...]) + b1_ref[...])
    o_ref[...] = _ln(_bdot(h, w2_ref[...]) + b2_ref[...]) + hn


def _node_update_pq_body(hn_ref, a1_ref, a2_ref, b1_ref_, b2_ref_, w1a_ref,
                         w1b1_ref, w1b2_ref, b1_ref, w2_ref, b2_ref,
                         w1s_ref, w1d_ref, o_ref, p_ref, q_ref):
    hn = hn_ref[...]
    ag1 = a1_ref[...] + b1_ref_[...]
    ag2 = a2_ref[...] + b2_ref_[...]
    h = jax.nn.silu(_bdot(hn, w1a_ref[...]) + _bdot(ag1, w1b1_ref[...])
                    + _bdot(ag2, w1b2_ref[...]) + b1_ref[...])
    hn2 = _ln(_bdot(h, w2_ref[...]) + b2_ref[...]) + hn
    o_ref[...] = hn2
    p_ref[...] = _pack_bf16(_bdot(hn2, w1s_ref[...]))
    q_ref[...] = _pack_bf16(_bdot(hn2, w1d_ref[...]))


def _node_update(h_node, aggs, w1a, w1b1, w1b2, b1, w2, b2,
                 w1s=None, w1d=None):
    base_specs = [
        pl.BlockSpec((_BN, _ND), lambda i: (i, 0)),
        pl.BlockSpec((_BN, _HC), lambda i: (i, 0)),
        pl.BlockSpec((_BN, _HC), lambda i: (i, 0)),
        pl.BlockSpec((_BN, _HC), lambda i: (i, 0)),
        pl.BlockSpec((_BN, _HC), lambda i: (i, 0)),
        pl.BlockSpec((_ND, _ND), lambda i: (0, 0)),
        pl.BlockSpec((_HC, _ND), lambda i: (0, 0)),
        pl.BlockSpec((_HC, _ND), lambda i: (0, 0)),
        pl.BlockSpec((1, _ND), lambda i: (0, 0)),
        pl.BlockSpec((_ND, _ND), lambda i: (0, 0)),
        pl.BlockSpec((1, _ND), lambda i: (0, 0)),
    ]
    args = [h_node] + list(aggs) + [w1a, w1b1, w1b2, b1.reshape(1, -1), w2,
                                    b2.reshape(1, -1)]
    if w1s is None:
        return pl.pallas_call(
            _node_update_body,
            grid=(_N // _BN,),
            in_specs=base_specs,
            out_specs=pl.BlockSpec((_BN, _ND), lambda i: (i, 0)),
            out_shape=jax.ShapeDtypeStruct((_N, _ND), jnp.float32),
        )(*args)
    return pl.pallas_call(
        _node_update_pq_body,
        grid=(_N // _BN,),
        in_specs=base_specs + [
            pl.BlockSpec((_ND, _ED), lambda i: (0, 0)),
            pl.BlockSpec((_ND, _ED), lambda i: (0, 0)),
        ],
        out_specs=[
            pl.BlockSpec((_BN, _ND), lambda i: (i, 0)),
            pl.BlockSpec((_BN, _ED // 2), lambda i: (i, 0)),
            pl.BlockSpec((_BN, _ED // 2), lambda i: (i, 0)),
        ],
        out_shape=[
            jax.ShapeDtypeStruct((_N, _ND), jnp.float32),
            jax.ShapeDtypeStruct((_N, _ED // 2), jnp.int32),
            jax.ShapeDtypeStruct((_N, _ED // 2), jnp.int32),
        ],
    )(*(args + [w1s, w1d]))


def _pq_body(hn_ref, w1s_ref, w1d_ref, p_ref, q_ref):
    hn = hn_ref[...]
    p_ref[...] = _pack_bf16(_bdot(hn, w1s_ref[...]))
    q_ref[...] = _pack_bf16(_bdot(hn, w1d_ref[...]))


def _pq(h_node, w1s, w1d):
    return pl.pallas_call(
        _pq_body,
        grid=(_N // _BN,),
        in_specs=[
            pl.BlockSpec((_BN, _ND), lambda i: (i, 0)),
            pl.BlockSpec((_ND, _ED), lambda i: (0, 0)),
            pl.BlockSpec((_ND, _ED), lambda i: (0, 0)),
        ],
        out_specs=[
            pl.BlockSpec((_BN, _ED // 2), lambda i: (i, 0)),
            pl.BlockSpec((_BN, _ED // 2), lambda i: (i, 0)),
        ],
        out_shape=[
            jax.ShapeDtypeStruct((_N, _ED // 2), jnp.int32),
            jax.ShapeDtypeStruct((_N, _ED // 2), jnp.int32),
        ],
    )(h_node, w1s, w1d)


def _decoder_body(hn_ref, w1_ref, b1_ref, w2_ref, b2_ref, o_ref):
    h = jax.nn.silu(_bdot(hn_ref[...], w1_ref[...]) + b1_ref[...])
    o_ref[...] = _bdot(h, w2_ref[...]) + b2_ref[...]


def _decoder(h_node, w1, b1, w2p, b2p):
    d_out = w2p.shape[1]
    return pl.pallas_call(
        _decoder_body,
        grid=(_N // _BN,),
        in_specs=[
            pl.BlockSpec((_BN, _ND), lambda i: (i, 0)),
            pl.BlockSpec((_ND, _ND), lambda i: (0, 0)),
            pl.BlockSpec((1, _ND), lambda i: (0, 0)),
            pl.BlockSpec((_ND, d_out), lambda i: (0, 0)),
            pl.BlockSpec((1, d_out), lambda i: (0, 0)),
        ],
        out_specs=pl.BlockSpec((_BN, d_out), lambda i: (i, 0)),
        out_shape=jax.ShapeDtypeStruct((_N, d_out), jnp.float32),
    )(h_node, w1, b1.reshape(1, -1), w2p, b2p.reshape(1, -1))


_INFO = plsc.get_sparse_core_info()
_NC = _INFO.num_cores
_NS = _INFO.num_subcores
_LN = _INFO.num_lanes
_NW = _NC * _NS

_GC = 128
_HC = _ED // _NC
_CO = 80
_NROWCH = _N // _CO
_RPT = -(-_NROWCH // _NS)
_HW = _ED // 2


def _sc_gather_sum(p, q, src, dst, e_start, e_count):
    mesh = plsc.VectorSubcoreMesh(core_axis_name="c", subcore_axis_name="s")
    nchunk = e_count // _GC
    gpw = -(-nchunk // _NW)
    mask_c = jnp.int32(-65536)
    half_c = jnp.int32(32768)

    @functools.partial(
        pl.kernel,
        mesh=mesh,
        out_type=jax.ShapeDtypeStruct((e_count, _HW), jnp.int32),
        scratch_types=[
            pltpu.VMEM((2, _GC), jnp.int32),
            pltpu.VMEM((2, _GC), jnp.int32),
            pltpu.VMEM((2, _GC, _HW), jnp.int32),
            pltpu.VMEM((2, _GC, _HW), jnp.int32),
            pltpu.VMEM((2, _GC, _HW), jnp.int32),
            pltpu.SemaphoreType.DMA,
            pltpu.SemaphoreType.DMA,
            pltpu.SemaphoreType.DMA,
            pltpu.SemaphoreType.DMA,
            pltpu.SemaphoreType.DMA,
            pltpu.SemaphoreType.DMA,
        ],
    )
    def k(p_hbm, q_hbm, src_hbm, dst_hbm, out_hbm, sidx, didx, bufa, bufb,
          obuf, sa0, sa1, sb0, sb1, so0, so1):
        wid = lax.axis_index("s") * _NC + lax.axis_index("c")
        sas = (sa0, sa1)
        sbs = (sb0, sb1)
        sos = (so0, so1)

        def issue(g, slot):
            chunk = g * _NW + wid

            @pl.when(chunk < nchunk)
            def _():
                eg = e_start + chunk * _GC
                pltpu.sync_copy(src_hbm.at[pl.ds(eg, _GC)], sidx.at[slot])
                pltpu.sync_copy(dst_hbm.at[pl.ds(eg, _GC)], didx.at[slot])
                pltpu.async_copy(p_hbm.at[sidx.at[slot]], bufa.at[slot],
                                 sas[slot])
                pltpu.async_copy(q_hbm.at[didx.at[slot]], bufb.at[slot],
                                 sbs[slot])

        def process(g, slot):
            chunk = g * _NW + wid

            @pl.when(chunk < nchunk)
            def _():
                e0 = chunk * _GC
                pltpu.make_async_copy(p_hbm.at[sidx.at[slot]],
                                      bufa.at[slot], sas[slot]).wait()
                pltpu.make_async_copy(q_hbm.at[didx.at[slot]],
                                      bufb.at[slot], sbs[slot]).wait()

                def add_row(r2, c2):
                    for v in range(2):
                      r = r2 * 2 + v
                      for u in range(_HW // _LN):
                        sl = pl.ds(u * _LN, _LN)
                        a = bufa[slot, r, sl]
                        b = bufb[slot, r, sl]
                        lo = (lax.bitcast_convert_type(
                                  lax.shift_left(a, 16), jnp.float32)
                              + lax.bitcast_convert_type(
                                  lax.shift_left(b, 16), jnp.float32))
                        hi = (lax.bitcast_convert_type(a & mask_c,
                                                       jnp.float32)
                              + lax.bitcast_convert_type(b & mask_c,
                                                         jnp.float32))
                        ulo = lax.bitcast_convert_type(lo, jnp.int32) + half_c
                        uhi = lax.bitcast_convert_type(hi, jnp.int32) + half_c
                        obuf[slot, r, sl] = (
                            lax.shift_right_logical(ulo, 16) | (uhi & mask_c))
                    return c2

                lax.fori_loop(0, _GC // 2, add_row, 0)
                pltpu.async_copy(obuf.at[slot], out_hbm.at[pl.ds(e0, _GC)],
                                 sos[slot])

        def drainw(g, slot):
            chunk = g * _NW + wid

            @pl.when((g >= 0) & (chunk < nchunk))
            def _():
                e0 = chunk * _GC
                pltpu.make_async_copy(obuf.at[slot],
                                      out_hbm.at[pl.ds(e0, _GC)],
                                      sos[slot]).wait()

        issue(0, 0)
        issue(1, 1)

        def step(j, carry):
            g0 = j * 2
            drainw(g0 - 2, 0)
            process(g0, 0)
            issue(g0 + 2, 0)
            drainw(g0 - 1, 1)
            process(g0 + 1, 1)
            issue(g0 + 3, 1)
            return carry

        nsteps = -(-gpw // 2)
        lax.fori_loop(0, nsteps, step, 0)
        drainw(nsteps * 2 - 2, 0)
        drainw(nsteps * 2 - 1, 1)

    return k(p, q, src, dst)


def _sc_segment_sum(he, dst, e_start):
    mesh = plsc.VectorSubcoreMesh(core_axis_name="c", subcore_axis_name="s")
    e_count = he.shape[0]
    nchunk = e_count // _GC
    cpt = -(-nchunk // _NS)

    @functools.partial(
        pl.kernel,
        mesh=mesh,
        out_type=[jax.ShapeDtypeStruct((_N, _HC), jnp.float32),
                  jax.ShapeDtypeStruct((_N, _HC), jnp.float32)],
        scratch_types=[
            pltpu.VMEM((2, _GC), jnp.int32),
            pltpu.VMEM((2, _GC, _HC), jnp.float32),
            pltpu.VMEM((_CO, _HC), jnp.float32),
            pltpu.VMEM_SHARED((_N, _HC), jnp.float32),
            pltpu.SemaphoreType.DMA,
            pltpu.SemaphoreType.DMA,
            pltpu.SemaphoreType.DMA,
            pltpu.SemaphoreType.DMA,
            pltpu.SemaphoreType.DMA,
            pltpu.SemaphoreType.DMA,
        ],
    )
    def k(he_hbm, dst_hbm, o1_hbm, o2_hbm, idx, data, cobuf, table,
          li0, li1, ld0, ld1, ss0, ss1):
        c = lax.axis_index("c")
        s = lax.axis_index("s")
        lis = (li0, li1)
        lds = (ld0, ld1)
        sss = (ss0, ss1)
        col0 = c * _HC

        def load(g, slot):
            chunk = g * _NS + s

            @pl.when(chunk < nchunk)
            def _():
                e0 = chunk * _GC
                pltpu.async_copy(dst_hbm.at[pl.ds(e_start + e0, _GC)],
                                 idx.at[slot], lis[slot])
                pltpu.async_copy(he_hbm.at[pl.ds(e0, _GC), pl.ds(col0, _HC)],
                                 data.at[slot], lds[slot])

        def scat(g, slot):
            chunk = g * _NS + s

            @pl.when(chunk < nchunk)
            def _():
                e0 = chunk * _GC
                pltpu.make_async_copy(dst_hbm.at[pl.ds(e_start + e0, _GC)],
                                      idx.at[slot], lis[slot]).wait()
                pltpu.make_async_copy(
                    he_hbm.at[pl.ds(e0, _GC), pl.ds(col0, _HC)],
                    data.at[slot], lds[slot]).wait()
                pltpu.async_copy(data.at[slot], table.at[idx.at[slot]],
                                 sss[slot], add=True)

        def drains(g, slot):
            chunk = g * _NS + s

            @pl.when(chunk < nchunk)
            def _():
                pltpu.make_async_copy(data.at[slot],
                                      table.at[idx.at[slot]],
                                      sss[slot]).wait()

        load(0, 0)
        load(1, 1)

        def zrow(r, carry):
            for u in range(_HC // _LN):
                cobuf[r, pl.ds(u * _LN, _LN)] = jnp.zeros((_LN,), jnp.float32)
            return carry

        lax.fori_loop(0, _CO, zrow, 0)

        def zchunk(j, carry):
            rc = j * _NS + s

            @pl.when(rc < _NROWCH)
            def _():
                pltpu.sync_copy(cobuf, table.at[pl.ds(rc * _CO, _CO)])

            return carry

        lax.fori_loop(0, _RPT, zchunk, 0)
        plsc.subcore_barrier()

        def step(j, carry):
            g0 = j * 2
            scat(g0, 0)
            drains(g0, 0)
            load(g0 + 2, 0)
            scat(g0 + 1, 1)
            drains(g0 + 1, 1)
            load(g0 + 3, 1)
            return carry

        lax.fori_loop(0, -(-cpt // 2), step, 0)
        plsc.subcore_barrier()

        def cochunk(j, carry):
            rc = j * _NS + s

            @pl.when(rc < _NROWCH)
            def _():
                pltpu.sync_copy(table.at[pl.ds(rc * _CO, _CO)], cobuf)

                @pl.when(c == 0)
                def _():
                    pltpu.sync_copy(cobuf, o1_hbm.at[pl.ds(rc * _CO, _CO)])

                @pl.when(c == 1)
                def _():
                    pltpu.sync_copy(cobuf, o2_hbm.at[pl.ds(rc * _CO, _CO)])

            return carry

        lax.fori_loop(0, _RPT, cochunk, 0)

    return k(he, dst)


def kernel(x, edge_index, edge_attr, params):
    src = edge_index[0]
    dst = edge_index[1]

    en = params["enc_n"]
    ee = params["enc_e"]
    h_node = _mlp_ln(x, en[0], en[1], en[2], en[3], _BN)
    he_a = _mlp_ln(edge_attr, ee[0], ee[1], ee[2], ee[3], _BE,
                   row_off=0, rows_out=_EH)
    he_b = _mlp_ln(edge_attr, ee[0], ee[1], ee[2], ee[3], _BE,
                   row_off=_EH, rows_out=_EH)

    convs = params["convs"]
    ew = [cp["edge"] for cp in convs]
    p32, q32 = _pq(h_node, ew[0][0][:_ND], ew[0][0][_ND:2 * _ND])
    for i, cp in enumerate(convs):
        w1, b1, w2, b2 = ew[i]
        w1e = w1[2 * _ND:]
        ga = _sc_gather_sum(p32, q32, src, dst, 0, _EH)
        gb = _sc_gather_sum(p32, q32, src, dst, _EH, _EH)
        he_a = _edge_update(ga, he_a, w1e, b1, w2, b2)
        agg_a = _sc_segment_sum(he_a, dst, 0)
        he_b = _edge_update(gb, he_b, w1e, b1, w2, b2)
        agg_b = _sc_segment_sum(he_b, dst, _EH)
        nw1, nb1, nw2, nb2 = cp["node"]
        aggs = (agg_a[0], agg_a[1], agg_b[0], agg_b[1])
        if i + 1 < len(convs):
            nxt = ew[i + 1][0]
            h_node, p32, q32 = _node_update(
                h_node, aggs, nw1[:_ND], nw1[_ND:_ND + _HC],
                nw1[_ND + _HC:], nb1, nw2, nb2,
                w1s=nxt[:_ND], w1d=nxt[_ND:2 * _ND])
        else:
            h_node = _node_update(h_node, aggs, nw1[:_ND],
                                  nw1[_ND:_ND + _HC], nw1[_ND + _HC:],
                                  nb1, nw2, nb2)

    ow1, ob1, ow2, ob2 = params["out"]
    d_out = ow2.shape[1]
    w2p = jnp.pad(ow2, ((0, 0), (0, 128 - d_out)))
    b2p = jnp.pad(ob2, (0, 128 - d_out))
    out = _decoder(h_node, ow1, ob1, w2p, b2p)
    return out[:, :d_out]


# speedup vs baseline: 4.8137x; 1.0005x over previous
import functools

import jax
import jax.numpy as jnp
from jax import lax
from jax.experimental import pallas as pl
from jax.experimental.pallas import tpu as pltpu
from jax.experimental.pallas import tpu_sc as plsc

_N = 10000
_E = 160000
_EH = _E // 2
_ND = 256
_ED = 256

_BN = 2000
_BE = 4000

_EPS = 1e-5


def _ln(y):
    mu = jnp.mean(y, axis=-1, keepdims=True)
    var = jnp.mean((y - mu) ** 2, axis=-1, keepdims=True)
    return (y - mu) * lax.rsqrt(var + _EPS)


def _bdot(a, b):
    return jnp.dot(a, b, preferred_element_type=jnp.float32)


def _pack_bf16(y):
    d2 = y.shape[1] // 2
    u = lax.bitcast_convert_type(y, jnp.uint32)
    rnd = (u + jnp.uint32(0x7FFF) + ((u >> 16) & jnp.uint32(1))) >> 16
    w = rnd[:, :d2] | (rnd[:, d2:] << 16)
    return lax.bitcast_convert_type(w, jnp.int32)


def _unpack_bf16(w):
    u = lax.bitcast_convert_type(w, jnp.uint32)
    ylo = lax.bitcast_convert_type(u << 16, jnp.float32)
    yhi = lax.bitcast_convert_type(u & jnp.uint32(0xFFFF0000), jnp.float32)
    return jnp.concatenate([ylo, yhi], axis=1)


def _mlp_ln_body(x_ref, w1_ref, b1_ref, w2_ref, b2_ref, o_ref):
    h = jax.nn.silu(_bdot(x_ref[...], w1_ref[...]) + b1_ref[...])
    o_ref[...] = _ln(_bdot(h, w2_ref[...]) + b2_ref[...])


def _mlp_ln(xin, w1, b1, w2, b2, block, row_off=0, rows_out=None):
    rows, d_in = xin.shape
    if rows_out is None:
        rows_out = rows
    hdim = w1.shape[1]
    d_out = w2.shape[1]
    off = row_off // block
    return pl.pallas_call(
        _mlp_ln_body,
        grid=(rows_out // block,),
        in_specs=[
            pl.BlockSpec((block, d_in), lambda i: (i + off, 0)),
            pl.BlockSpec((d_in, hdim), lambda i: (0, 0)),
            pl.BlockSpec((1, hdim), lambda i: (0, 0)),
            pl.BlockSpec((hdim, d_out), lambda i: (0, 0)),
            pl.BlockSpec((1, d_out), lambda i: (0, 0)),
        ],
        out_specs=pl.BlockSpec((block, d_out), lambda i: (i, 0)),
        out_shape=jax.ShapeDtypeStruct((rows_out, d_out), jnp.float32),
    )(xin, w1, b1.reshape(1, -1), w2, b2.reshape(1, -1))


def _edge_update_body(g_ref, he_ref, w1e_ref, b1_ref, w2_ref, b2_ref,
                      o_ref):
    he = he_ref[...]
    g = _unpack_bf16(g_ref[...])
    h = jax.nn.silu(g + _bdot(he, w1e_ref[...]) + b1_ref[...])
    o_ref[...] = _ln(_bdot(h, w2_ref[...]) + b2_ref[...]) + he


def _edge_update(g, h_edge, w1e, b1, w2, b2):
    rows = g.shape[0]
    return pl.pallas_call(
        _edge_update_body,
        grid=(rows // _BE,),
        in_specs=[
            pl.BlockSpec((_BE, _ED // 2), lambda i: (i, 0)),
            pl.BlockSpec((_BE, _ED), lambda i: (i, 0)),
            pl.BlockSpec((_ED, _ED), lambda i: (0, 0)),
            pl.BlockSpec((1, _ED), lambda i: (0, 0)),
            pl.BlockSpec((_ED, _ED), lambda i: (0, 0)),
            pl.BlockSpec((1, _ED), lambda i: (0, 0)),
        ],
        out_specs=pl.BlockSpec((_BE, _ED), lambda i: (i, 0)),
        out_shape=jax.ShapeDtypeStruct((rows, _ED), jnp.float32),
    )(g, h_edge, w1e, b1.reshape(1, -1), w2, b2.reshape(1, -1))


def _node_update_body(hn_ref, a1_ref, a2_ref, b1_ref_, b2_ref_, w1a_ref,
                      w1b1_ref, w1b2_ref, b1_ref, w2_ref, b2_ref, o_ref):
    hn = hn_ref[...]
    ag1 = a1_ref[...] + b1_ref_[...]
    ag2 = a2_ref[...] + b2_ref_[...]
    h = jax.nn.silu(_bdot(hn, w1a_ref[...]) + _bdot(ag1, w1b1_ref[...])
                    + _bdot(ag2, w1b2_ref[...]) + b1_ref[...])
    o_ref[...] = _ln(_bdot(h, w2_ref[...]) + b2_ref[...]) + hn


def _node_update_pq_body(hn_ref, a1_ref, a2_ref, b1_ref_, b2_ref_, w1a_ref,
                         w1b1_ref, w1b2_ref, b1_ref, w2_ref, b2_ref,
                         w1s_ref, w1d_ref, o_ref, p_ref, q_ref):
    hn = hn_ref[...]
    ag1 = a1_ref[...] + b1_ref_[...]
    ag2 = a2_ref[...] + b2_ref_[...]
    h = jax.nn.silu(_bdot(hn, w1a_ref[...]) + _bdot(ag1, w1b1_ref[...])
                    + _bdot(ag2, w1b2_ref[...]) + b1_ref[...])
    hn2 = _ln(_bdot(h, w2_ref[...]) + b2_ref[...]) + hn
    o_ref[...] = hn2
    p_ref[...] = _pack_bf16(_bdot(hn2, w1s_ref[...]))
    q_ref[...] = _pack_bf16(_bdot(hn2, w1d_ref[...]))


def _node_update(h_node, aggs, w1a, w1b1, w1b2, b1, w2, b2,
                 w1s=None, w1d=None):
    base_specs = [
        pl.BlockSpec((_BN, _ND), lambda i: (i, 0)),
        pl.BlockSpec((_BN, _HC), lambda i: (i, 0)),
        pl.BlockSpec((_BN, _HC), lambda i: (i, 0)),
        pl.BlockSpec((_BN, _HC), lambda i: (i, 0)),
        pl.BlockSpec((_BN, _HC), lambda i: (i, 0)),
        pl.BlockSpec((_ND, _ND), lambda i: (0, 0)),
        pl.BlockSpec((_HC, _ND), lambda i: (0, 0)),
        pl.BlockSpec((_HC, _ND), lambda i: (0, 0)),
        pl.BlockSpec((1, _ND), lambda i: (0, 0)),
        pl.BlockSpec((_ND, _ND), lambda i: (0, 0)),
        pl.BlockSpec((1, _ND), lambda i: (0, 0)),
    ]
    args = [h_node] + list(aggs) + [w1a, w1b1, w1b2, b1.reshape(1, -1), w2,
                                    b2.reshape(1, -1)]
    if w1s is None:
        return pl.pallas_call(
            _node_update_body,
            grid=(_N // _BN,),
            in_specs=base_specs,
            out_specs=pl.BlockSpec((_BN, _ND), lambda i: (i, 0)),
            out_shape=jax.ShapeDtypeStruct((_N, _ND), jnp.float32),
        )(*args)
    return pl.pallas_call(
        _node_update_pq_body,
        grid=(_N // _BN,),
        in_specs=base_specs + [
            pl.BlockSpec((_ND, _ED), lambda i: (0, 0)),
            pl.BlockSpec((_ND, _ED), lambda i: (0, 0)),
        ],
        out_specs=[
            pl.BlockSpec((_BN, _ND), lambda i: (i, 0)),
            pl.BlockSpec((_BN, _ED // 2), lambda i: (i, 0)),
            pl.BlockSpec((_BN, _ED // 2), lambda i: (i, 0)),
        ],
        out_shape=[
            jax.ShapeDtypeStruct((_N, _ND), jnp.float32),
            jax.ShapeDtypeStruct((_N, _ED // 2), jnp.int32),
            jax.ShapeDtypeStruct((_N, _ED // 2), jnp.int32),
        ],
    )(*(args + [w1s, w1d]))


def _pq_body(hn_ref, w1s_ref, w1d_ref, p_ref, q_ref):
    hn = hn_ref[...]
    p_ref[...] = _pack_bf16(_bdot(hn, w1s_ref[...]))
    q_ref[...] = _pack_bf16(_bdot(hn, w1d_ref[...]))


def _pq(h_node, w1s, w1d):
    return pl.pallas_call(
        _pq_body,
        grid=(_N // _BN,),
        in_specs=[
            pl.BlockSpec((_BN, _ND), lambda i: (i, 0)),
            pl.BlockSpec((_ND, _ED), lambda i: (0, 0)),
            pl.BlockSpec((_ND, _ED), lambda i: (0, 0)),
        ],
        out_specs=[
            pl.BlockSpec((_BN, _ED // 2), lambda i: (i, 0)),
            pl.BlockSpec((_BN, _ED // 2), lambda i: (i, 0)),
        ],
        out_shape=[
            jax.ShapeDtypeStruct((_N, _ED // 2), jnp.int32),
            jax.ShapeDtypeStruct((_N, _ED // 2), jnp.int32),
        ],
    )(h_node, w1s, w1d)


def _decoder_body(hn_ref, w1_ref, b1_ref, w2_ref, b2_ref, o_ref):
    h = jax.nn.silu(_bdot(hn_ref[...], w1_ref[...]) + b1_ref[...])
    o_ref[...] = _bdot(h, w2_ref[...]) + b2_ref[...]


def _decoder(h_node, w1, b1, w2p, b2p):
    d_out = w2p.shape[1]
    return pl.pallas_call(
        _decoder_body,
        grid=(_N // _BN,),
        in_specs=[
            pl.BlockSpec((_BN, _ND), lambda i: (i, 0)),
            pl.BlockSpec((_ND, _ND), lambda i: (0, 0)),
            pl.BlockSpec((1, _ND), lambda i: (0, 0)),
            pl.BlockSpec((_ND, d_out), lambda i: (0, 0)),
            pl.BlockSpec((1, d_out), lambda i: (0, 0)),
        ],
        out_specs=pl.BlockSpec((_BN, d_out), lambda i: (i, 0)),
        out_shape=jax.ShapeDtypeStruct((_N, d_out), jnp.float32),
    )(h_node, w1, b1.reshape(1, -1), w2p, b2p.reshape(1, -1))


_INFO = plsc.get_sparse_core_info()
_NC = _INFO.num_cores
_NS = _INFO.num_subcores
_LN = _INFO.num_lanes
_NW = _NC * _NS

_GC = 128
_GCS = 128
_HC = _ED // _NC
_CO = 80
_NROWCH = _N // _CO
_RPT = -(-_NROWCH // _NS)
_HW = _ED // 2


def _sc_gather_sum(p, q, src, dst, e_start, e_count):
    mesh = plsc.VectorSubcoreMesh(core_axis_name="c", subcore_axis_name="s")
    nchunk = e_count // _GC
    gpw = -(-nchunk // _NW)
    mask_c = jnp.int32(-65536)
    half_c = jnp.int32(32768)

    @functools.partial(
        pl.kernel,
        mesh=mesh,
        out_type=jax.ShapeDtypeStruct((e_count, _HW), jnp.int32),
        scratch_types=[
            pltpu.VMEM((2, _GC), jnp.int32),
            pltpu.VMEM((2, _GC), jnp.int32),
            pltpu.VMEM((2, _GC, _HW), jnp.int32),
            pltpu.VMEM((2, _GC, _HW), jnp.int32),
            pltpu.VMEM((2, _GC, _HW), jnp.int32),
            pltpu.SemaphoreType.DMA,
            pltpu.SemaphoreType.DMA,
            pltpu.SemaphoreType.DMA,
            pltpu.SemaphoreType.DMA,
            pltpu.SemaphoreType.DMA,
            pltpu.SemaphoreType.DMA,
        ],
    )
    def k(p_hbm, q_hbm, src_hbm, dst_hbm, out_hbm, sidx, didx, bufa, bufb,
          obuf, sa0, sa1, sb0, sb1, so0, so1):
        wid = lax.axis_index("s") * _NC + lax.axis_index("c")
        sas = (sa0, sa1)
        sbs = (sb0, sb1)
        sos = (so0, so1)

        def issue(g, slot):
            chunk = g * _NW + wid

            @pl.when(chunk < nchunk)
            def _():
                eg = e_start + chunk * _GC
                pltpu.sync_copy(src_hbm.at[pl.ds(eg, _GC)], sidx.at[slot])
                pltpu.sync_copy(dst_hbm.at[pl.ds(eg, _GC)], didx.at[slot])
                pltpu.async_copy(p_hbm.at[sidx.at[slot]], bufa.at[slot],
                                 sas[slot])
                pltpu.async_copy(q_hbm.at[didx.at[slot]], bufb.at[slot],
                                 sbs[slot])

        def process(g, slot):
            chunk = g * _NW + wid

            @pl.when(chunk < nchunk)
            def _():
                e0 = chunk * _GC
                pltpu.make_async_copy(p_hbm.at[sidx.at[slot]],
                                      bufa.at[slot], sas[slot]).wait()
                pltpu.make_async_copy(q_hbm.at[didx.at[slot]],
                                      bufb.at[slot], sbs[slot]).wait()

                def add_row(r2, c2):
                    for v in range(2):
                      r = r2 * 2 + v
                      for u in range(_HW // _LN):
                        sl = pl.ds(u * _LN, _LN)
                        a = bufa[slot, r, sl]
                        b = bufb[slot, r, sl]
                        lo = (lax.bitcast_convert_type(
                                  lax.shift_left(a, 16), jnp.float32)
                              + lax.bitcast_convert_type(
                                  lax.shift_left(b, 16), jnp.float32))
                        hi = (lax.bitcast_convert_type(a & mask_c,
                                                       jnp.float32)
                              + lax.bitcast_convert_type(b & mask_c,
                                                         jnp.float32))
                        ulo = lax.bitcast_convert_type(lo, jnp.int32) + half_c
                        uhi = lax.bitcast_convert_type(hi, jnp.int32) + half_c
                        obuf[slot, r, sl] = (
                            lax.shift_right_logical(ulo, 16) | (uhi & mask_c))
                    return c2

                lax.fori_loop(0, _GC // 2, add_row, 0)
                pltpu.async_copy(obuf.at[slot], out_hbm.at[pl.ds(e0, _GC)],
                                 sos[slot])

        def drainw(g, slot):
            chunk = g * _NW + wid

            @pl.when((g >= 0) & (chunk < nchunk))
            def _():
                e0 = chunk * _GC
                pltpu.make_async_copy(obuf.at[slot],
                                      out_hbm.at[pl.ds(e0, _GC)],
                                      sos[slot]).wait()

        issue(0, 0)
        issue(1, 1)

        def step(j, carry):
            g0 = j * 2
            drainw(g0 - 2, 0)
            process(g0, 0)
            issue(g0 + 2, 0)
            drainw(g0 - 1, 1)
            process(g0 + 1, 1)
            issue(g0 + 3, 1)
            return carry

        nsteps = -(-gpw // 2)
        lax.fori_loop(0, nsteps, step, 0)
        drainw(nsteps * 2 - 2, 0)
        drainw(nsteps * 2 - 1, 1)

    return k(p, q, src, dst)


def _sc_segment_sum(he, dst, e_start):
    mesh = plsc.VectorSubcoreMesh(core_axis_name="c", subcore_axis_name="s")
    e_count = he.shape[0]
    nchunk = e_count // _GCS
    cpt = -(-nchunk // _NS)

    @functools.partial(
        pl.kernel,
        mesh=mesh,
        out_type=[jax.ShapeDtypeStruct((_N, _HC), jnp.float32),
                  jax.ShapeDtypeStruct((_N, _HC), jnp.float32)],
        scratch_types=[
            pltpu.VMEM((2, _GCS), jnp.int32),
            pltpu.VMEM((2, _GCS, _HC), jnp.float32),
            pltpu.VMEM((_CO, _HC), jnp.float32),
            pltpu.VMEM_SHARED((_N, _HC), jnp.float32),
            pltpu.SemaphoreType.DMA,
            pltpu.SemaphoreType.DMA,
            pltpu.SemaphoreType.DMA,
            pltpu.SemaphoreType.DMA,
            pltpu.SemaphoreType.DMA,
            pltpu.SemaphoreType.DMA,
        ],
    )
    def k(he_hbm, dst_hbm, o1_hbm, o2_hbm, idx, data, cobuf, table,
          li0, li1, ld0, ld1, ss0, ss1):
        c = lax.axis_index("c")
        s = lax.axis_index("s")
        lis = (li0, li1)
        lds = (ld0, ld1)
        sss = (ss0, ss1)
        col0 = c * _HC

        def load(g, slot):
            chunk = g * _NS + s

            @pl.when(chunk < nchunk)
            def _():
                e0 = chunk * _GCS
                pltpu.async_copy(dst_hbm.at[pl.ds(e_start + e0, _GCS)],
                                 idx.at[slot], lis[slot])
                pltpu.async_copy(he_hbm.at[pl.ds(e0, _GCS), pl.ds(col0, _HC)],
                                 data.at[slot], lds[slot])

        def scat(g, slot):
            chunk = g * _NS + s

            @pl.when(chunk < nchunk)
            def _():
                e0 = chunk * _GCS
                pltpu.make_async_copy(dst_hbm.at[pl.ds(e_start + e0, _GCS)],
                                      idx.at[slot], lis[slot]).wait()
                pltpu.make_async_copy(
                    he_hbm.at[pl.ds(e0, _GCS), pl.ds(col0, _HC)],
                    data.at[slot], lds[slot]).wait()
                pltpu.async_copy(data.at[slot], table.at[idx.at[slot]],
                                 sss[slot], add=True)

        def drains(g, slot):
            chunk = g * _NS + s

            @pl.when(chunk < nchunk)
            def _():
                pltpu.make_async_copy(data.at[slot],
                                      table.at[idx.at[slot]],
                                      sss[slot]).wait()

        load(0, 0)
        load(1, 1)

        def zrow(r, carry):
            for u in range(_HC // _LN):
                cobuf[r, pl.ds(u * _LN, _LN)] = jnp.zeros((_LN,), jnp.float32)
            return carry

        lax.fori_loop(0, _CO, zrow, 0)

        def zchunk(j, carry):
            rc = j * _NS + s

            @pl.when(rc < _NROWCH)
            def _():
                pltpu.sync_copy(cobuf, table.at[pl.ds(rc * _CO, _CO)])

            return carry

        lax.fori_loop(0, _RPT, zchunk, 0)
        plsc.subcore_barrier()

        def step(j, carry):
            g0 = j * 2
            scat(g0, 0)
            drains(g0, 0)
            load(g0 + 2, 0)
            scat(g0 + 1, 1)
            drains(g0 + 1, 1)
            load(g0 + 3, 1)
            return carry

        lax.fori_loop(0, -(-cpt // 2), step, 0)
        plsc.subcore_barrier()

        def cochunk(j, carry):
            rc = j * _NS + s

            @pl.when(rc < _NROWCH)
            def _():
                pltpu.sync_copy(table.at[pl.ds(rc * _CO, _CO)], cobuf)

                @pl.when(c == 0)
                def _():
                    pltpu.sync_copy(cobuf, o1_hbm.at[pl.ds(rc * _CO, _CO)])

                @pl.when(c == 1)
                def _():
                    pltpu.sync_copy(cobuf, o2_hbm.at[pl.ds(rc * _CO, _CO)])

            return carry

        lax.fori_loop(0, _RPT, cochunk, 0)

    return k(he, dst)


def kernel(x, edge_index, edge_attr, params):
    src = edge_index[0]
    dst = edge_index[1]

    en = params["enc_n"]
    ee = params["enc_e"]
    h_node = _mlp_ln(x, en[0], en[1], en[2], en[3], _BN)
    he_a = _mlp_ln(edge_attr, ee[0], ee[1], ee[2], ee[3], _BE,
                   row_off=0, rows_out=_EH)
    he_b = _mlp_ln(edge_attr, ee[0], ee[1], ee[2], ee[3], _BE,
                   row_off=_EH, rows_out=_EH)

    convs = params["convs"]
    ew = [cp["edge"] for cp in convs]
    p32, q32 = _pq(h_node, ew[0][0][:_ND], ew[0][0][_ND:2 * _ND])
    for i, cp in enumerate(convs):
        w1, b1, w2, b2 = ew[i]
        w1e = w1[2 * _ND:]
        ga = _sc_gather_sum(p32, q32, src, dst, 0, _EH)
        gb = _sc_gather_sum(p32, q32, src, dst, _EH, _EH)
        he_a = _edge_update(ga, he_a, w1e, b1, w2, b2)
        agg_a = _sc_segment_sum(he_a, dst, 0)
        he_b = _edge_update(gb, he_b, w1e, b1, w2, b2)
        agg_b = _sc_segment_sum(he_b, dst, _EH)
        nw1, nb1, nw2, nb2 = cp["node"]
        aggs = (agg_a[0], agg_a[1], agg_b[0], agg_b[1])
        if i + 1 < len(convs):
            nxt = ew[i + 1][0]
            h_node, p32, q32 = _node_update(
                h_node, aggs, nw1[:_ND], nw1[_ND:_ND + _HC],
                nw1[_ND + _HC:], nb1, nw2, nb2,
                w1s=nxt[:_ND], w1d=nxt[_ND:2 * _ND])
        else:
            h_node = _node_update(h_node, aggs, nw1[:_ND],
                                  nw1[_ND:_ND + _HC], nw1[_ND + _HC:],
                                  nb1, nw2, nb2)

    ow1, ob1, ow2, ob2 = params["out"]
    d_out = ow2.shape[1]
    w2p = jnp.pad(ow2, ((0, 0), (0, 128 - d_out)))
    b2p = jnp.pad(ob2, (0, 128 - d_out))
    out = _decoder(h_node, ow1, ob1, w2p, b2p)
    return out[:, :d_out]


# speedup vs baseline: 4.9049x; 1.0189x over previous
import functools

import jax
import jax.numpy as jnp
from jax import lax
from jax.experimental import pallas as pl
from jax.experimental.pallas import tpu as pltpu
from jax.experimental.pallas import tpu_sc as plsc

_N = 10000
_E = 160000
_EH = _E // 2
_ND = 256
_ED = 256

_BN = 2000
_BE = 4000

_EPS = 1e-5


def _ln(y):
    mu = jnp.mean(y, axis=-1, keepdims=True)
    var = jnp.mean((y - mu) ** 2, axis=-1, keepdims=True)
    return (y - mu) * lax.rsqrt(var + _EPS)


def _bdot(a, b):
    return jnp.dot(a, b, preferred_element_type=jnp.float32)


def _pack_bf16(y):
    d2 = y.shape[1] // 2
    u = lax.bitcast_convert_type(y, jnp.uint32)
    rnd = (u + jnp.uint32(0x7FFF) + ((u >> 16) & jnp.uint32(1))) >> 16
    w = rnd[:, :d2] | (rnd[:, d2:] << 16)
    return lax.bitcast_convert_type(w, jnp.int32)


def _unpack_bf16(w):
    u = lax.bitcast_convert_type(w, jnp.uint32)
    ylo = lax.bitcast_convert_type(u << 16, jnp.float32)
    yhi = lax.bitcast_convert_type(u & jnp.uint32(0xFFFF0000), jnp.float32)
    return jnp.concatenate([ylo, yhi], axis=1)


def _mlp_ln_body(x_ref, w1_ref, b1_ref, w2_ref, b2_ref, o_ref):
    h = jax.nn.silu(_bdot(x_ref[...], w1_ref[...]) + b1_ref[...])
    o_ref[...] = _ln(_bdot(h, w2_ref[...]) + b2_ref[...])


def _mlp_ln(xin, w1, b1, w2, b2, block, row_off=0, rows_out=None):
    rows, d_in = xin.shape
    if rows_out is None:
        rows_out = rows
    hdim = w1.shape[1]
    d_out = w2.shape[1]
    off = row_off // block
    return pl.pallas_call(
        _mlp_ln_body,
        grid=(rows_out // block,),
        in_specs=[
            pl.BlockSpec((block, d_in), lambda i: (i + off, 0)),
            pl.BlockSpec((d_in, hdim), lambda i: (0, 0)),
            pl.BlockSpec((1, hdim), lambda i: (0, 0)),
            pl.BlockSpec((hdim, d_out), lambda i: (0, 0)),
            pl.BlockSpec((1, d_out), lambda i: (0, 0)),
        ],
        out_specs=pl.BlockSpec((block, d_out), lambda i: (i, 0)),
        out_shape=jax.ShapeDtypeStruct((rows_out, d_out), jnp.float32),
    )(xin, w1, b1.reshape(1, -1), w2, b2.reshape(1, -1))


def _edge_update_body(g_ref, he_ref, w1e_ref, b1_ref, w2_ref, b2_ref,
                      o_ref):
    he = he_ref[...]
    g = _unpack_bf16(g_ref[...])
    h = jax.nn.silu(g + _bdot(he, w1e_ref[...]) + b1_ref[...])
    o_ref[...] = _ln(_bdot(h, w2_ref[...]) + b2_ref[...]) + he


def _edge_update(g, h_edge, w1e, b1, w2, b2):
    rows = g.shape[0]
    return pl.pallas_call(
        _edge_update_body,
        grid=(rows // _BE,),
        in_specs=[
            pl.BlockSpec((_BE, _ED // 2), lambda i: (i, 0)),
            pl.BlockSpec((_BE, _ED), lambda i: (i, 0)),
            pl.BlockSpec((_ED, _ED), lambda i: (0, 0)),
            pl.BlockSpec((1, _ED), lambda i: (0, 0)),
            pl.BlockSpec((_ED, _ED), lambda i: (0, 0)),
            pl.BlockSpec((1, _ED), lambda i: (0, 0)),
        ],
        out_specs=pl.BlockSpec((_BE, _ED), lambda i: (i, 0)),
        out_shape=jax.ShapeDtypeStruct((rows, _ED), jnp.float32),
    )(g, h_edge, w1e, b1.reshape(1, -1), w2, b2.reshape(1, -1))


def _node_update_body(hn_ref, a1_ref, a2_ref, b1_ref_, b2_ref_, w1a_ref,
                      w1b1_ref, w1b2_ref, b1_ref, w2_ref, b2_ref, o_ref):
    hn = hn_ref[...]
    ag1 = a1_ref[...] + b1_ref_[...]
    ag2 = a2_ref[...] + b2_ref_[...]
    h = jax.nn.silu(_bdot(hn, w1a_ref[...]) + _bdot(ag1, w1b1_ref[...])
                    + _bdot(ag2, w1b2_ref[...]) + b1_ref[...])
    o_ref[...] = _ln(_bdot(h, w2_ref[...]) + b2_ref[...]) + hn


def _node_update_pq_body(hn_ref, a1_ref, a2_ref, b1_ref_, b2_ref_, w1a_ref,
                         w1b1_ref, w1b2_ref, b1_ref, w2_ref, b2_ref,
                         w1s_ref, w1d_ref, o_ref, p_ref, q_ref):
    hn = hn_ref[...]
    ag1 = a1_ref[...] + b1_ref_[...]
    ag2 = a2_ref[...] + b2_ref_[...]
    h = jax.nn.silu(_bdot(hn, w1a_ref[...]) + _bdot(ag1, w1b1_ref[...])
                    + _bdot(ag2, w1b2_ref[...]) + b1_ref[...])
    hn2 = _ln(_bdot(h, w2_ref[...]) + b2_ref[...]) + hn
    o_ref[...] = hn2
    p_ref[...] = _pack_bf16(_bdot(hn2, w1s_ref[...]))
    q_ref[...] = _pack_bf16(_bdot(hn2, w1d_ref[...]))


def _node_update(h_node, aggs, w1a, w1b1, w1b2, b1, w2, b2,
                 w1s=None, w1d=None):
    base_specs = [
        pl.BlockSpec((_BN, _ND), lambda i: (i, 0)),
        pl.BlockSpec((_BN, _HC), lambda i: (i, 0)),
        pl.BlockSpec((_BN, _HC), lambda i: (i, 0)),
        pl.BlockSpec((_BN, _HC), lambda i: (i, 0)),
        pl.BlockSpec((_BN, _HC), lambda i: (i, 0)),
        pl.BlockSpec((_ND, _ND), lambda i: (0, 0)),
        pl.BlockSpec((_HC, _ND), lambda i: (0, 0)),
        pl.BlockSpec((_HC, _ND), lambda i: (0, 0)),
        pl.BlockSpec((1, _ND), lambda i: (0, 0)),
        pl.BlockSpec((_ND, _ND), lambda i: (0, 0)),
        pl.BlockSpec((1, _ND), lambda i: (0, 0)),
    ]
    args = [h_node] + list(aggs) + [w1a, w1b1, w1b2, b1.reshape(1, -1), w2,
                                    b2.reshape(1, -1)]
    if w1s is None:
        return pl.pallas_call(
            _node_update_body,
            grid=(_N // _BN,),
            in_specs=base_specs,
            out_specs=pl.BlockSpec((_BN, _ND), lambda i: (i, 0)),
            out_shape=jax.ShapeDtypeStruct((_N, _ND), jnp.float32),
        )(*args)
    return pl.pallas_call(
        _node_update_pq_body,
        grid=(_N // _BN,),
        in_specs=base_specs + [
            pl.BlockSpec((_ND, _ED), lambda i: (0, 0)),
            pl.BlockSpec((_ND, _ED), lambda i: (0, 0)),
        ],
        out_specs=[
            pl.BlockSpec((_BN, _ND), lambda i: (i, 0)),
            pl.BlockSpec((_BN, _ED // 2), lambda i: (i, 0)),
            pl.BlockSpec((_BN, _ED // 2), lambda i: (i, 0)),
        ],
        out_shape=[
            jax.ShapeDtypeStruct((_N, _ND), jnp.float32),
            jax.ShapeDtypeStruct((_N, _ED // 2), jnp.int32),
            jax.ShapeDtypeStruct((_N, _ED // 2), jnp.int32),
        ],
    )(*(args + [w1s, w1d]))


def _pq_body(hn_ref, w1s_ref, w1d_ref, p_ref, q_ref):
    hn = hn_ref[...]
    p_ref[...] = _pack_bf16(_bdot(hn, w1s_ref[...]))
    q_ref[...] = _pack_bf16(_bdot(hn, w1d_ref[...]))


def _pq(h_node, w1s, w1d):
    return pl.pallas_call(
        _pq_body,
        grid=(_N // _BN,),
        in_specs=[
            pl.BlockSpec((_BN, _ND), lambda i: (i, 0)),
            pl.BlockSpec((_ND, _ED), lambda i: (0, 0)),
            pl.BlockSpec((_ND, _ED), lambda i: (0, 0)),
        ],
        out_specs=[
            pl.BlockSpec((_BN, _ED // 2), lambda i: (i, 0)),
            pl.BlockSpec((_BN, _ED // 2), lambda i: (i, 0)),
        ],
        out_shape=[
            jax.ShapeDtypeStruct((_N, _ED // 2), jnp.int32),
            jax.ShapeDtypeStruct((_N, _ED // 2), jnp.int32),
        ],
    )(h_node, w1s, w1d)


def _decoder_body(hn_ref, w1_ref, b1_ref, w2_ref, b2_ref, o_ref):
    h = jax.nn.silu(_bdot(hn_ref[...], w1_ref[...]) + b1_ref[...])
    o_ref[...] = _bdot(h, w2_ref[...]) + b2_ref[...]


def _decoder(h_node, w1, b1, w2p, b2p):
    d_out = w2p.shape[1]
    return pl.pallas_call(
        _decoder_body,
        grid=(_N // _BN,),
        in_specs=[
            pl.BlockSpec((_BN, _ND), lambda i: (i, 0)),
            pl.BlockSpec((_ND, _ND), lambda i: (0, 0)),
            pl.BlockSpec((1, _ND), lambda i: (0, 0)),
            pl.BlockSpec((_ND, d_out), lambda i: (0, 0)),
            pl.BlockSpec((1, d_out), lambda i: (0, 0)),
        ],
        out_specs=pl.BlockSpec((_BN, d_out), lambda i: (i, 0)),
        out_shape=jax.ShapeDtypeStruct((_N, d_out), jnp.float32),
    )(h_node, w1, b1.reshape(1, -1), w2p, b2p.reshape(1, -1))


_INFO = plsc.get_sparse_core_info()
_NC = _INFO.num_cores
_NS = _INFO.num_subcores
_LN = _INFO.num_lanes
_NW = _NC * _NS

_GC = 128
_GCS = 128
_HC = _ED // _NC
_CO = 80
_NROWCH = _N // _CO
_RPT = -(-_NROWCH // _NS)
_HW = _ED // 2


def _sc_gather_sum(p, q, src2, dst2, e_start, e_count):
    mesh = plsc.VectorSubcoreMesh(core_axis_name="c", subcore_axis_name="s")
    nchunk = e_count // _GC
    gpw = -(-nchunk // _NW)
    rbase = e_start // _GC
    mask_c = jnp.int32(-65536)
    half_c = jnp.int32(32768)

    @functools.partial(
        pl.kernel,
        mesh=mesh,
        out_type=jax.ShapeDtypeStruct((e_count, _HW), jnp.int32),
        scratch_types=[
            pltpu.VMEM((gpw + 12, _GC), jnp.int32),
            pltpu.VMEM((gpw + 12, _GC), jnp.int32),
            pltpu.VMEM((2, _GC, _HW), jnp.int32),
            pltpu.VMEM((2, _GC, _HW), jnp.int32),
            pltpu.VMEM((2, _GC, _HW), jnp.int32),
            pltpu.SemaphoreType.DMA,
            pltpu.SemaphoreType.DMA,
            pltpu.SemaphoreType.DMA,
            pltpu.SemaphoreType.DMA,
            pltpu.SemaphoreType.DMA,
            pltpu.SemaphoreType.DMA,
        ],
    )
    def k(p_hbm, q_hbm, src_hbm, dst_hbm, out_hbm, sidx, didx, bufa, bufb,
          obuf, sa0, sa1, sb0, sb1, so0, so1):
        wid = lax.axis_index("s") * _NC + lax.axis_index("c")
        row0 = jnp.minimum(wid * gpw, nchunk - gpw)
        grow = rbase + row0
        grd = pl.multiple_of(grow & jnp.int32(-8), 8)
        off = grow - grd
        sas = (sa0, sa1)
        sbs = (sb0, sb1)
        sos = (so0, so1)

        pltpu.sync_copy(src_hbm.at[pl.ds(grd, gpw + 12)], sidx)
        pltpu.sync_copy(dst_hbm.at[pl.ds(grd, gpw + 12)], didx)

        def issue(j, slot):
            jj = off + j
            pltpu.async_copy(p_hbm.at[sidx.at[jj]], bufa.at[slot], sas[slot])
            pltpu.async_copy(q_hbm.at[didx.at[jj]], bufb.at[slot], sbs[slot])

        def process(j, slot):
            jj = off + j
            e0 = (row0 + j) * _GC
            pltpu.make_async_copy(p_hbm.at[sidx.at[jj]],
                                  bufa.at[slot], sas[slot]).wait()
            pltpu.make_async_copy(q_hbm.at[didx.at[jj]],
                                  bufb.at[slot], sbs[slot]).wait()

            def add_row(r2, c2):
                for v in range(2):
                  r = r2 * 2 + v
                  for u in range(_HW // _LN):
                    sl = pl.ds(u * _LN, _LN)
                    a = bufa[slot, r, sl]
                    b = bufb[slot, r, sl]
                    lo = (lax.bitcast_convert_type(
                              lax.shift_left(a, 16), jnp.float32)
                          + lax.bitcast_convert_type(
                              lax.shift_left(b, 16), jnp.float32))
                    hi = (lax.bitcast_convert_type(a & mask_c,
                                                   jnp.float32)
                          + lax.bitcast_convert_type(b & mask_c,
                                                     jnp.float32))
                    ulo = lax.bitcast_convert_type(lo, jnp.int32) + half_c
                    uhi = lax.bitcast_convert_type(hi, jnp.int32) + half_c
                    obuf[slot, r, sl] = (
                        lax.shift_right_logical(ulo, 16) | (uhi & mask_c))
                return c2

            lax.fori_loop(0, _GC // 2, add_row, 0)
            pltpu.async_copy(obuf.at[slot], out_hbm.at[pl.ds(e0, _GC)],
                             sos[slot])

        def drainw(j, slot):
            e0 = (row0 + j) * _GC
            pltpu.make_async_copy(obuf.at[slot],
                                  out_hbm.at[pl.ds(e0, _GC)],
                                  sos[slot]).wait()

        issue(0, 0)
        issue(1, 1)

        def step(j2, carry):
            j0 = j2 * 2

            @pl.when(j2 > 0)
            def _():
                drainw(j0 - 2, 0)

            process(j0, 0)

            @pl.when(j0 + 2 < gpw)
            def _():
                issue(j0 + 2, 0)

            @pl.when(j2 > 0)
            def _():
                drainw(j0 - 1, 1)

            process(j0 + 1, 1)

            @pl.when(j0 + 3 < gpw)
            def _():
                issue(j0 + 3, 1)

            return carry

        lax.fori_loop(0, gpw // 2, step, 0)
        drainw(gpw - 2, 0)
        drainw(gpw - 1, 1)

    return k(p, q, src2, dst2)


def _sc_segment_sum(he, dst, e_start):
    mesh = plsc.VectorSubcoreMesh(core_axis_name="c", subcore_axis_name="s")
    e_count = he.shape[0]
    nchunk = e_count // _GCS
    cpt = -(-nchunk // _NS)

    @functools.partial(
        pl.kernel,
        mesh=mesh,
        out_type=[jax.ShapeDtypeStruct((_N, _HC), jnp.float32),
                  jax.ShapeDtypeStruct((_N, _HC), jnp.float32)],
        scratch_types=[
            pltpu.VMEM((2, _GCS), jnp.int32),
            pltpu.VMEM((2, _GCS, _HC), jnp.float32),
            pltpu.VMEM((_CO, _HC), jnp.float32),
            pltpu.VMEM_SHARED((_N, _HC), jnp.float32),
            pltpu.SemaphoreType.DMA,
            pltpu.SemaphoreType.DMA,
            pltpu.SemaphoreType.DMA,
            pltpu.SemaphoreType.DMA,
            pltpu.SemaphoreType.DMA,
            pltpu.SemaphoreType.DMA,
        ],
    )
    def k(he_hbm, dst_hbm, o1_hbm, o2_hbm, idx, data, cobuf, table,
          li0, li1, ld0, ld1, ss0, ss1):
        c = lax.axis_index("c")
        s = lax.axis_index("s")
        lis = (li0, li1)
        lds = (ld0, ld1)
        sss = (ss0, ss1)
        col0 = c * _HC

        def load(g, slot):
            chunk = g * _NS + s

            @pl.when(chunk < nchunk)
            def _():
                e0 = chunk * _GCS
                pltpu.async_copy(dst_hbm.at[pl.ds(e_start + e0, _GCS)],
                                 idx.at[slot], lis[slot])
                pltpu.async_copy(he_hbm.at[pl.ds(e0, _GCS), pl.ds(col0, _HC)],
                                 data.at[slot], lds[slot])

        def scat(g, slot):
            chunk = g * _NS + s

            @pl.when(chunk < nchunk)
            def _():
                e0 = chunk * _GCS
                pltpu.make_async_copy(dst_hbm.at[pl.ds(e_start + e0, _GCS)],
                                      idx.at[slot], lis[slot]).wait()
                pltpu.make_async_copy(
                    he_hbm.at[pl.ds(e0, _GCS), pl.ds(col0, _HC)],
                    data.at[slot], lds[slot]).wait()
                pltpu.async_copy(data.at[slot], table.at[idx.at[slot]],
                                 sss[slot], add=True)

        def drains(g, slot):
            chunk = g * _NS + s

            @pl.when(chunk < nchunk)
            def _():
                pltpu.make_async_copy(data.at[slot],
                                      table.at[idx.at[slot]],
                                      sss[slot]).wait()

        load(0, 0)
        load(1, 1)

        def zrow(r, carry):
            for u in range(_HC // _LN):
                cobuf[r, pl.ds(u * _LN, _LN)] = jnp.zeros((_LN,), jnp.float32)
            return carry

        lax.fori_loop(0, _CO, zrow, 0)

        def zchunk(j, carry):
            rc = j * _NS + s

            @pl.when(rc < _NROWCH)
            def _():
                pltpu.sync_copy(cobuf, table.at[pl.ds(rc * _CO, _CO)])

            return carry

        lax.fori_loop(0, _RPT, zchunk, 0)
        plsc.subcore_barrier()

        def step(j, carry):
            g0 = j * 2
            scat(g0, 0)
            drains(g0, 0)
            load(g0 + 2, 0)
            scat(g0 + 1, 1)
            drains(g0 + 1, 1)
            load(g0 + 3, 1)
            return carry

        lax.fori_loop(0, -(-cpt // 2), step, 0)
        plsc.subcore_barrier()

        def cochunk(j, carry):
            rc = j * _NS + s

            @pl.when(rc < _NROWCH)
            def _():
                pltpu.sync_copy(table.at[pl.ds(rc * _CO, _CO)], cobuf)

                @pl.when(c == 0)
                def _():
                    pltpu.sync_copy(cobuf, o1_hbm.at[pl.ds(rc * _CO, _CO)])

                @pl.when(c == 1)
                def _():
                    pltpu.sync_copy(cobuf, o2_hbm.at[pl.ds(rc * _CO, _CO)])

            return carry

        lax.fori_loop(0, _RPT, cochunk, 0)

    return k(he, dst)


def kernel(x, edge_index, edge_attr, params):
    src = edge_index[0]
    dst = edge_index[1]
    src2 = jnp.pad(src.reshape(_E // _GC, _GC), ((0, 14), (0, 0)))
    dst2 = jnp.pad(dst.reshape(_E // _GC, _GC), ((0, 14), (0, 0)))

    en = params["enc_n"]
    ee = params["enc_e"]
    h_node = _mlp_ln(x, en[0], en[1], en[2], en[3], _BN)
    he_a = _mlp_ln(edge_attr, ee[0], ee[1], ee[2], ee[3], _BE,
                   row_off=0, rows_out=_EH)
    he_b = _mlp_ln(edge_attr, ee[0], ee[1], ee[2], ee[3], _BE,
                   row_off=_EH, rows_out=_EH)

    convs = params["convs"]
    ew = [cp["edge"] for cp in convs]
    p32, q32 = _pq(h_node, ew[0][0][:_ND], ew[0][0][_ND:2 * _ND])
    for i, cp in enumerate(convs):
        w1, b1, w2, b2 = ew[i]
        w1e = w1[2 * _ND:]
        ga = _sc_gather_sum(p32, q32, src2, dst2, 0, _EH)
        gb = _sc_gather_sum(p32, q32, src2, dst2, _EH, _EH)
        he_a = _edge_update(ga, he_a, w1e, b1, w2, b2)
        agg_a = _sc_segment_sum(he_a, dst, 0)
        he_b = _edge_update(gb, he_b, w1e, b1, w2, b2)
        agg_b = _sc_segment_sum(he_b, dst, _EH)
        nw1, nb1, nw2, nb2 = cp["node"]
        aggs = (agg_a[0], agg_a[1], agg_b[0], agg_b[1])
        if i + 1 < len(convs):
            nxt = ew[i + 1][0]
            h_node, p32, q32 = _node_update(
                h_node, aggs, nw1[:_ND], nw1[_ND:_ND + _HC],
                nw1[_ND + _HC:], nb1, nw2, nb2,
                w1s=nxt[:_ND], w1d=nxt[_ND:2 * _ND])
        else:
            h_node = _node_update(h_node, aggs, nw1[:_ND],
                                  nw1[_ND:_ND + _HC], nw1[_ND + _HC:],
                                  nb1, nw2, nb2)

    ow1, ob1, ow2, ob2 = params["out"]
    d_out = ow2.shape[1]
    w2p = jnp.pad(ow2, ((0, 0), (0, 128 - d_out)))
    b2p = jnp.pad(ob2, (0, 128 - d_out))
    out = _decoder(h_node, ow1, ob1, w2p, b2p)
    return out[:, :d_out]
